# baseline (speedup 1.0000x reference)
import functools
import jax
import jax.numpy as jnp
from jax import lax
from jax.experimental import pallas as pl
from jax.experimental.pallas import tpu as pltpu
from jax.experimental.pallas import tpu_sc as plsc

D = 64
NC = 2
NS = 16
NW = NC * NS
B = 4096 * 50
B_PER_W = B // NW
CH = 800
N_CH = B_PER_W // CH
SCALE = 8.0

_mesh = plsc.VectorSubcoreMesh(core_axis_name="c", subcore_axis_name="s")


@functools.partial(
    pl.kernel,
    out_type=jax.ShapeDtypeStruct((B, D), jnp.bfloat16),
    mesh=_mesh,
    scratch_types=[
        pltpu.VMEM((CH,), jnp.int32),
        pltpu.VMEM((CH,), jnp.int32),
        pltpu.VMEM((CH, D), jnp.bfloat16),
        pltpu.VMEM((CH, D), jnp.bfloat16),
        pltpu.SemaphoreType.DMA,
        pltpu.SemaphoreType.DMA,
        pltpu.SemaphoreType.DMA,
    ],
    compiler_params=pltpu.CompilerParams(use_tc_tiling_on_sc=False),
)
def _emb_lookup(x_hbm, table_hbm, out_hbm, idx_a, idx_b, rows_a, rows_b,
                gsem_a, gsem_b, wsem):
    wid = lax.axis_index("s") * NC + lax.axis_index("c")
    base = wid * B_PER_W

    def load_idx(ci, idx_v):
        pltpu.sync_copy(x_hbm.at[pl.ds(base + ci * CH, CH)], idx_v)

    def start_gather(idx_v, rows_v, gsem):
        return pltpu.async_copy(table_hbm.at[idx_v], rows_v, gsem)

    def finish(ci, rows_v, gdesc, first):
        gdesc.wait()

        def scale_body(r, c):
            for k in range(D // 32):
                sl = pl.ds(k * 32, 32)
                rows_v[r, sl] = rows_v[r, sl] * SCALE
            return c

        lax.fori_loop(0, CH, scale_body, 0)
        if not first:
            pltpu.make_async_copy(
                rows_v, out_hbm.at[pl.ds(base, CH)], wsem).wait()
        pltpu.async_copy(rows_v, out_hbm.at[pl.ds(base + ci * CH, CH)], wsem)

    load_idx(0, idx_a)
    g = start_gather(idx_a, rows_a, gsem_a)
    for ci in range(N_CH):
        nxt = None
        if ci + 1 < N_CH:
            idx_n = idx_b if ci % 2 == 0 else idx_a
            rows_n = rows_b if ci % 2 == 0 else rows_a
            gsem_n = gsem_b if ci % 2 == 0 else gsem_a
            load_idx(ci + 1, idx_n)
            nxt = start_gather(idx_n, rows_n, gsem_n)
        rows_c = rows_a if ci % 2 == 0 else rows_b
        finish(ci, rows_c, g, first=ci < 2)
        g = nxt
    pltpu.make_async_copy(rows_a, out_hbm.at[pl.ds(base, CH)], wsem).wait()
    pltpu.make_async_copy(rows_b, out_hbm.at[pl.ds(base, CH)], wsem).wait()


def kernel(x, embedding_weight):
    xf = x.reshape(-1).astype(jnp.int32)
    table_bf = embedding_weight.astype(jnp.bfloat16)
    out = _emb_lookup(xf, table_bf)
    return out.astype(jnp.float32).reshape(x.shape[0], x.shape[1], D)


# speedup vs baseline: 1.2782x; 1.2782x over previous
import functools
import jax
import jax.numpy as jnp
from jax import lax
from jax.experimental import pallas as pl
from jax.experimental.pallas import tpu as pltpu
from jax.experimental.pallas import tpu_sc as plsc

D = 64
NC = 2
NS = 16
NW = NC * NS
B = 4096 * 50
B_PER_W = B // NW
CH = 256
N_CH = B_PER_W // CH
SCALE = 8.0

_mesh = plsc.VectorSubcoreMesh(core_axis_name="c", subcore_axis_name="s")


def _half_select(rows_v, oc_v, par_v, g0):
    pv = par_v[pl.ds(2 * g0, 16)]
    for j in range(8):
        g = g0 + j
        for e in range(2):
            f = 2 * j + e
            off = pv[f] * D
            for k in range(D // 16):
                sl_src = pl.ds(off + k * 16, 16)
                sl_dst = pl.ds(e * D + k * 16, 16)
                oc_v[g, sl_dst] = rows_v[2 * g + e, sl_src] * SCALE


@functools.partial(
    pl.kernel,
    out_type=jax.ShapeDtypeStruct((B // 2, 2 * D), jnp.float32),
    mesh=_mesh,
    scratch_types=[
        pltpu.VMEM((CH,), jnp.int32),
        pltpu.VMEM((CH,), jnp.int32),
        pltpu.VMEM((CH,), jnp.int32),
        pltpu.VMEM((CH,), jnp.int32),
        pltpu.VMEM((CH,), jnp.int32),
        pltpu.VMEM((CH,), jnp.int32),
        pltpu.VMEM((CH, 2 * D), jnp.float32),
        pltpu.VMEM((CH, 2 * D), jnp.float32),
        pltpu.VMEM((CH // 2, 2 * D), jnp.float32),
        pltpu.VMEM((CH // 2, 2 * D), jnp.float32),
        pltpu.SemaphoreType.DMA,
        pltpu.SemaphoreType.DMA,
        pltpu.SemaphoreType.DMA,
    ],
    compiler_params=pltpu.CompilerParams(use_tc_tiling_on_sc=True),
)
def _emb_lookup(x_hbm, table2_hbm, out_hbm, idx_a, idx_b, p_a, p_b, par_a,
                par_b, rows_a, rows_b, oc_a, oc_b, gsem_a, gsem_b, wsem):
    wid = lax.axis_index("s") * NC + lax.axis_index("c")
    base = pl.multiple_of(wid * B_PER_W, 256)

    def prep(ci, idx_v, p_v, par_v, gsem, rows_v):
        pltpu.sync_copy(x_hbm.at[pl.ds(base + ci * CH, CH)], idx_v)

        def split(v, c):
            sl = pl.ds(v * 16, 16)
            x16 = idx_v[sl]
            p_v[sl] = lax.shift_right_logical(x16, 1)
            par_v[sl] = lax.bitwise_and(x16, 1)
            return c

        lax.fori_loop(0, CH // 16, split, 0)
        return pltpu.async_copy(table2_hbm.at[p_v], rows_v, gsem)

    def finish(ci, rows_v, par_v, oc_v, gdesc, first):
        gdesc.wait()
        if not first:
            pltpu.make_async_copy(
                oc_v,
                out_hbm.at[pl.ds(pl.multiple_of(base // 2, 128), CH // 2)],
                wsem).wait()

        def build(t, c):
            _half_select(rows_v, oc_v, par_v, t * 8)
            return c

        lax.fori_loop(0, CH // 16, build, 0)
        off2 = pl.multiple_of((base + ci * CH) // 2, 128)
        pltpu.async_copy(oc_v, out_hbm.at[pl.ds(off2, CH // 2)], wsem)

    g = prep(0, idx_a, p_a, par_a, gsem_a, rows_a)
    for ci in range(N_CH):
        nxt = None
        if ci + 1 < N_CH:
            if ci % 2 == 0:
                nxt = prep(ci + 1, idx_b, p_b, par_b, gsem_b, rows_b)
            else:
                nxt = prep(ci + 1, idx_a, p_a, par_a, gsem_a, rows_a)
        rows_c = rows_a if ci % 2 == 0 else rows_b
        par_c = par_a if ci % 2 == 0 else par_b
        oc_c = oc_a if ci % 2 == 0 else oc_b
        finish(ci, rows_c, par_c, oc_c, g, first=ci < 2)
        g = nxt
    half = pl.multiple_of(base // 2, 128)
    pltpu.make_async_copy(oc_a, out_hbm.at[pl.ds(half, CH // 2)], wsem).wait()
    pltpu.make_async_copy(oc_b, out_hbm.at[pl.ds(half, CH // 2)], wsem).wait()


def kernel(x, embedding_weight):
    xf = x.reshape(-1).astype(jnp.int32)
    table2 = embedding_weight.reshape(500000, 128)
    out2 = _emb_lookup(xf, table2)
    return out2.reshape(x.shape[0], x.shape[1], D)


# speedup vs baseline: 1.4278x; 1.1170x over previous
import functools
import jax
import jax.numpy as jnp
from jax import lax
from jax.experimental import pallas as pl
from jax.experimental.pallas import tpu as pltpu
from jax.experimental.pallas import tpu_sc as plsc

D = 64
NC = 2
NS = 16
NW = NC * NS
B = 4096 * 50
B_PER_W = B // NW
CH = 800
N_CH = B_PER_W // CH
SCALE = 8.0

_mesh = plsc.VectorSubcoreMesh(core_axis_name="c", subcore_axis_name="s")


@functools.partial(
    pl.kernel,
    out_type=jax.ShapeDtypeStruct((B, D), jnp.float32),
    mesh=_mesh,
    scratch_types=[
        pltpu.VMEM((CH,), jnp.int32),
        pltpu.VMEM((CH, D), jnp.float32),
        pltpu.SemaphoreType.DMA,
    ],
    compiler_params=pltpu.CompilerParams(use_tc_tiling_on_sc=False),
)
def _emb_lookup(x_hbm, table_hbm, out_hbm, idx_v, rows_v, sem):
    wid = lax.axis_index("s") * NC + lax.axis_index("c")
    base = wid * B_PER_W

    def chunk_body(ci, carry):
        off = base + ci * CH
        pltpu.sync_copy(x_hbm.at[pl.ds(off, CH)], idx_v)
        pltpu.async_copy(table_hbm.at[idx_v], rows_v, sem).wait()

        def scale_body(r, c):
            for k in range(D // 16):
                rows_v[r, pl.ds(k * 16, 16)] = rows_v[r, pl.ds(k * 16, 16)] * SCALE
            return c

        lax.fori_loop(0, CH, scale_body, 0)
        pltpu.sync_copy(rows_v, out_hbm.at[pl.ds(off, CH)])
        return carry

    lax.fori_loop(0, N_CH, chunk_body, 0)


def kernel(x, embedding_weight):
    xf = x.astype(jnp.int32).T.reshape(-1)
    out = _emb_lookup(xf, embedding_weight)
    return out.reshape(x.shape[1], x.shape[0], D).transpose(1, 0, 2)


# speedup vs baseline: 1.4801x; 1.0366x over previous
import functools
import jax
import jax.numpy as jnp
from jax import lax
from jax.experimental import pallas as pl
from jax.experimental.pallas import tpu as pltpu
from jax.experimental.pallas import tpu_sc as plsc

D = 64
NC = 2
NS = 16
NW = NC * NS
B = 4096 * 50
B_PER_W = B // NW
CH = 800
N_CH = B_PER_W // CH
SCALE = 8.0

_mesh = plsc.VectorSubcoreMesh(core_axis_name="c", subcore_axis_name="s")


@functools.partial(
    pl.kernel,
    out_type=jax.ShapeDtypeStruct((B, D), jnp.float32),
    mesh=_mesh,
    scratch_types=[
        pltpu.VMEM((CH,), jnp.int32),
        pltpu.VMEM((CH,), jnp.int32),
        pltpu.VMEM((CH, D), jnp.float32),
        pltpu.VMEM((CH, D), jnp.float32),
        pltpu.SemaphoreType.DMA,
        pltpu.SemaphoreType.DMA,
        pltpu.SemaphoreType.DMA,
    ],
    compiler_params=pltpu.CompilerParams(use_tc_tiling_on_sc=False),
)
def _emb_lookup(x_hbm, table_hbm, out_hbm, idx_a, idx_b, rows_a, rows_b,
                gsem_a, gsem_b, wsem):
    wid = lax.axis_index("s") * NC + lax.axis_index("c")
    base = wid * B_PER_W

    def prep(ci, idx_v, rows_v, gsem):
        pltpu.sync_copy(x_hbm.at[pl.ds(base + ci * CH, CH)], idx_v)
        return pltpu.async_copy(table_hbm.at[idx_v], rows_v, gsem)

    def finish(ci, rows_v, gdesc, first):
        gdesc.wait()
        if not first:
            pltpu.make_async_copy(
                rows_v, out_hbm.at[pl.ds(base, CH)], wsem).wait()

        def scale_body(r, c):
            for k in range(D // 16):
                rows_v[r, pl.ds(k * 16, 16)] = rows_v[r, pl.ds(k * 16, 16)] * SCALE
            return c

        lax.fori_loop(0, CH, scale_body, 0)
        pltpu.async_copy(rows_v, out_hbm.at[pl.ds(base + ci * CH, CH)], wsem)

    g = prep(0, idx_a, rows_a, gsem_a)
    for ci in range(N_CH):
        nxt = None
        if ci + 1 < N_CH:
            if ci % 2 == 0:
                nxt = prep(ci + 1, idx_b, rows_b, gsem_b)
            else:
                nxt = prep(ci + 1, idx_a, rows_a, gsem_a)
        rows_c = rows_a if ci % 2 == 0 else rows_b
        finish(ci, rows_c, g, first=ci < 2)
        g = nxt
    pltpu.make_async_copy(rows_a, out_hbm.at[pl.ds(base, CH)], wsem).wait()
    pltpu.make_async_copy(rows_b, out_hbm.at[pl.ds(base, CH)], wsem).wait()


def kernel(x, embedding_weight):
    xf = x.astype(jnp.int32).T.reshape(-1)
    out = _emb_lookup(xf, embedding_weight)
    return out.reshape(x.shape[1], x.shape[0], D).transpose(1, 0, 2)
